# baseline (device time: 37612 ns/iter reference)
import jax
import jax.numpy as jnp
from jax import lax
from jax.experimental import pallas as pl
from jax.experimental.pallas import tpu as pltpu

N = 16
B, SQ, SKV = 2, 256, 256
HQ, DH = 64, 64
HLOC = HQ // N
DM = 512
HID = HLOC * DH
ROWS = B * SQ
CH = ROWS // N


def kernel(x, Wq, K_ext, V_ext, Wo):
    pos = lax.axis_index("i")

    xf = x.reshape(ROWS, DM)
    wq = lax.dynamic_slice(Wq, (0, pos * HID), (DM, HID))
    wo = lax.dynamic_slice(Wo, (pos * HID, 0), (HID, DM))
    kt = jnp.transpose(K_ext, (0, 2, 3, 1))
    vf = V_ext.reshape(B * SKV, HID)

    def body(x_ref, wq_ref, kt_ref, v_ref, wo_ref, out_ref,
             p_ref, ctx_ref, rs_ref, rs_send, rs_recv, ag_send, ag_recv):
        me = lax.axis_index("i")

        barrier = pltpu.get_barrier_semaphore()
        for j in range(N - 1):
            t = (me + 1 + j) % N
            pl.semaphore_signal(
                barrier, inc=1,
                device_id=(t,), device_id_type=pl.DeviceIdType.MESH,
            )
        pl.semaphore_wait(barrier, N - 1)

        q = jnp.dot(x_ref[...], wq_ref[...],
                    preferred_element_type=jnp.float32)
        qi = lax.broadcasted_iota(jnp.int32, (SQ, SKV), 0)
        ki = lax.broadcasted_iota(jnp.int32, (SQ, SKV), 1)
        mask = (jnp.abs(qi - ki) <= 128) | (ki < 32) | (qi < 32)
        neg = jnp.float32(-1e9)
        for b in range(B):
            for h in range(HLOC):
                qbh = q[b * SQ:(b + 1) * SQ, h * DH:(h + 1) * DH]
                s = jnp.dot(qbh, kt_ref[b, h],
                            preferred_element_type=jnp.float32) * 0.125
                s = jnp.where(mask, s, neg)
                m = jnp.max(s, axis=1, keepdims=True)
                w = jnp.exp(s - m)
                w = w / jnp.sum(w, axis=1, keepdims=True)
                vbh = v_ref[b * SKV:(b + 1) * SKV, h * DH:(h + 1) * DH]
                ctx_ref[b * SQ:(b + 1) * SQ, h * DH:(h + 1) * DH] = jnp.dot(
                    w, vbh, preferred_element_type=jnp.float32)
        p_ref[...] = jnp.dot(ctx_ref[...], wo_ref[...],
                             preferred_element_type=jnp.float32)

        sends1 = []
        for j in range(N - 1):
            t = (me + 1 + j) % N
            rdma = pltpu.make_async_remote_copy(
                src_ref=p_ref.at[pl.ds(t * CH, CH), :],
                dst_ref=rs_ref.at[me],
                send_sem=rs_send.at[t],
                recv_sem=rs_recv.at[me],
                device_id=(t,),
                device_id_type=pl.DeviceIdType.MESH,
            )
            rdma.start()
            sends1.append(rdma)

        acc = p_ref[pl.ds(me * CH, CH), :]
        for j in range(N - 1):
            src = (me + 1 + j) % N
            recv = pltpu.make_async_remote_copy(
                src_ref=rs_ref.at[src],
                dst_ref=rs_ref.at[src],
                send_sem=rs_send.at[src],
                recv_sem=rs_recv.at[src],
                device_id=(src,),
                device_id_type=pl.DeviceIdType.MESH,
            )
            recv.wait_recv()
            acc = acc + rs_ref[src]
        out_ref[pl.ds(me * CH, CH), :] = acc

        sends2 = []
        for j in range(N - 1):
            t = (me + 1 + j) % N
            rdma = pltpu.make_async_remote_copy(
                src_ref=out_ref.at[pl.ds(me * CH, CH), :],
                dst_ref=out_ref.at[pl.ds(me * CH, CH), :],
                send_sem=ag_send.at[t],
                recv_sem=ag_recv.at[me],
                device_id=(t,),
                device_id_type=pl.DeviceIdType.MESH,
            )
            rdma.start()
            sends2.append(rdma)

        for j in range(N - 1):
            src = (me + 1 + j) % N
            recv = pltpu.make_async_remote_copy(
                src_ref=out_ref.at[pl.ds(src * CH, CH), :],
                dst_ref=out_ref.at[pl.ds(src * CH, CH), :],
                send_sem=ag_send.at[src],
                recv_sem=ag_recv.at[src],
                device_id=(src,),
                device_id_type=pl.DeviceIdType.MESH,
            )
            recv.wait_recv()

        for r in sends1:
            r.wait_send()
        for r in sends2:
            r.wait_send()

    out = pl.pallas_call(
        body,
        out_shape=jax.ShapeDtypeStruct((ROWS, DM), jnp.float32),
        in_specs=[pl.BlockSpec(memory_space=pltpu.VMEM)] * 5,
        out_specs=pl.BlockSpec(memory_space=pltpu.VMEM),
        scratch_shapes=[
            pltpu.VMEM((ROWS, DM), jnp.float32),
            pltpu.VMEM((ROWS, HID), jnp.float32),
            pltpu.VMEM((N, CH, DM), jnp.float32),
            pltpu.SemaphoreType.DMA((N,)),
            pltpu.SemaphoreType.DMA((N,)),
            pltpu.SemaphoreType.DMA((N,)),
            pltpu.SemaphoreType.DMA((N,)),
        ],
        compiler_params=pltpu.CompilerParams(collective_id=0),
    )(xf, wq, kt, vf, wo)

    return out.reshape(B, SQ, DM)


# device time: 27459 ns/iter; 1.3698x vs baseline; 1.3698x over previous
import jax
import jax.numpy as jnp
from jax import lax
from jax.experimental import pallas as pl
from jax.experimental.pallas import tpu as pltpu

N = 16
B, SQ, SKV = 2, 256, 256
HQ, DH = 64, 64
HLOC = HQ // N
DM = 512
HID = HLOC * DH
ROWS = B * SQ
CH = ROWS // N
CPB = SQ // CH


def kernel(x, Wq, K_ext, V_ext, Wo):
    pos = lax.axis_index("i")

    xf = x.reshape(ROWS, DM)
    wq = lax.dynamic_slice(Wq, (0, pos * HID), (DM, HID)) * 0.125
    wo = lax.dynamic_slice(Wo, (pos * HID, 0), (HID, DM))
    kt = jnp.transpose(K_ext, (0, 2, 3, 1))
    vf = V_ext.reshape(B * SKV, HID)

    def body(x_ref, wq_ref, kt_ref, v_ref, wo_ref, out_ref,
             p16_ref, ctx_ref, rs_ref, ag_ref,
             rs_send, rs_recv, ag_send, ag_recv):
        me = lax.axis_index("i")

        barrier = pltpu.get_barrier_semaphore()
        for j in range(N - 1):
            t = (me + 1 + j) % N
            pl.semaphore_signal(
                barrier, inc=1,
                device_id=(t,), device_id_type=pl.DeviceIdType.MESH,
            )

        q = jnp.dot(x_ref[...], wq_ref[...],
                    preferred_element_type=jnp.float32)
        qi = lax.broadcasted_iota(jnp.int32, (SQ, SKV), 0)
        ki = lax.broadcasted_iota(jnp.int32, (SQ, SKV), 1)
        mask = (jnp.abs(qi - ki) <= 128) | (ki < 32) | (qi < 32)
        bias = jnp.where(mask, jnp.float32(0.0), jnp.float32(-1e9))

        sends1 = []
        for b in range(B):
            for h in range(HLOC):
                qbh = q[b * SQ:(b + 1) * SQ, h * DH:(h + 1) * DH]
                s = jnp.dot(qbh, kt_ref[b, h],
                            preferred_element_type=jnp.float32)
                w = jnp.exp(s + bias)
                recip = 1.0 / jnp.sum(w, axis=1, keepdims=True)
                vbh = v_ref[b * SKV:(b + 1) * SKV, h * DH:(h + 1) * DH]
                ctx_ref[b * SQ:(b + 1) * SQ, h * DH:(h + 1) * DH] = jnp.dot(
                    w, vbh, preferred_element_type=jnp.float32) * recip
            pb = jnp.dot(ctx_ref[b * SQ:(b + 1) * SQ, :], wo_ref[...],
                         preferred_element_type=jnp.float32)
            p16_ref[b * SQ:(b + 1) * SQ, :] = pb.astype(jnp.bfloat16)

            if b == 0:
                pl.semaphore_wait(barrier, N - 1)

            for j in range(CPB):
                c = b * CPB + (me + 1 + j) % CPB
                rdma = pltpu.make_async_remote_copy(
                    src_ref=p16_ref.at[pl.ds(c * CH, CH), :],
                    dst_ref=rs_ref.at[me],
                    send_sem=rs_send.at[c],
                    recv_sem=rs_recv.at[me],
                    device_id=(c,),
                    device_id_type=pl.DeviceIdType.MESH,
                )
                rdma.start()
                sends1.append(rdma)

        acc = None
        for j in [0] + sorted(range(1, N), key=lambda d: min(d, N - d)):
            src = (me + j) % N
            recv = pltpu.make_async_remote_copy(
                src_ref=rs_ref.at[src],
                dst_ref=rs_ref.at[src],
                send_sem=rs_send.at[src],
                recv_sem=rs_recv.at[src],
                device_id=(src,),
                device_id_type=pl.DeviceIdType.MESH,
            )
            recv.wait_recv()
            term = rs_ref[src].astype(jnp.float32)
            acc = term if acc is None else acc + term
        p16_ref[pl.ds(me * CH, CH), :] = acc.astype(jnp.bfloat16)

        sends2 = []
        for j in range(N):
            t = (me + 1 + j) % N
            rdma = pltpu.make_async_remote_copy(
                src_ref=p16_ref.at[pl.ds(me * CH, CH), :],
                dst_ref=ag_ref.at[pl.ds(me * CH, CH), :],
                send_sem=ag_send.at[t],
                recv_sem=ag_recv.at[me],
                device_id=(t,),
                device_id_type=pl.DeviceIdType.MESH,
            )
            rdma.start()
            sends2.append(rdma)

        for j in [0] + sorted(range(1, N), key=lambda d: min(d, N - d)):
            src = (me + j) % N
            recv = pltpu.make_async_remote_copy(
                src_ref=p16_ref.at[pl.ds(src * CH, CH), :],
                dst_ref=ag_ref.at[pl.ds(src * CH, CH), :],
                send_sem=ag_send.at[src],
                recv_sem=ag_recv.at[src],
                device_id=(src,),
                device_id_type=pl.DeviceIdType.MESH,
            )
            recv.wait_recv()
            out_ref[pl.ds(src * CH, CH), :] = ag_ref[
                pl.ds(src * CH, CH), :].astype(jnp.float32)

        for r in sends1:
            r.wait_send()
        for r in sends2:
            r.wait_send()

    out = pl.pallas_call(
        body,
        out_shape=jax.ShapeDtypeStruct((ROWS, DM), jnp.float32),
        in_specs=[pl.BlockSpec(memory_space=pltpu.VMEM)] * 5,
        out_specs=pl.BlockSpec(memory_space=pltpu.VMEM),
        scratch_shapes=[
            pltpu.VMEM((ROWS, DM), jnp.bfloat16),
            pltpu.VMEM((ROWS, HID), jnp.float32),
            pltpu.VMEM((N, CH, DM), jnp.bfloat16),
            pltpu.VMEM((ROWS, DM), jnp.bfloat16),
            pltpu.SemaphoreType.DMA((N,)),
            pltpu.SemaphoreType.DMA((N,)),
            pltpu.SemaphoreType.DMA((N,)),
            pltpu.SemaphoreType.DMA((N,)),
        ],
        compiler_params=pltpu.CompilerParams(collective_id=0),
    )(xf, wq, kt, vf, wo)

    return out.reshape(B, SQ, DM)
